# baseline (speedup 1.0000x reference)
import functools

import jax
import jax.numpy as jnp
import numpy as np
from jax import lax
from jax.experimental import pallas as pl
from jax.experimental.pallas import tpu as pltpu
from jax.experimental.pallas import tpu_sc as plsc

N = 10000
E = 320000
D = 128

NC = 2
NS = 16
NW = NC * NS

CH = 128
CPW = 79
E_PAD = NW * CH * CPW
N_PAD = 10112
STRIPE = N_PAD // NS

_mesh = plsc.VectorSubcoreMesh(core_axis_name="c", subcore_axis_name="s")


_CNT_AHEAD = 8

@functools.partial(
    pl.kernel,
    out_type=jax.ShapeDtypeStruct((NC, N_PAD * 16), jnp.float32),
    mesh=_mesh,
    scratch_types=[
        pltpu.VMEM((CPW, CH), jnp.int32),
        pltpu.VMEM((CPW, CH), jnp.int32),
        pltpu.VMEM((CPW, CH), jnp.int32),
        pltpu.VMEM((CH,), jnp.float32),
        pltpu.VMEM_SHARED((N_PAD * 16,), jnp.float32),
        pltpu.SemaphoreType.DMA,
    ],
)
def _sc_counts(dst_hbm, k_hbm, z16_hbm, out_hbm, dst_v, k_v, fidx_v, ones_v,
               cnt_sh, sem):
    cid = lax.axis_index("c")
    sid = lax.axis_index("s")
    wid = sid * NC + cid
    seg = (N_PAD * 16) // NS
    el0 = sid * seg
    pltpu.sync_copy(dst_hbm.at[wid], dst_v)
    pltpu.sync_copy(k_hbm.at[wid], k_v)
    pltpu.sync_copy(z16_hbm.at[pl.ds(el0, seg)], cnt_sh.at[pl.ds(el0, seg)])
    ones = jnp.ones((16,), jnp.float32)
    for g in range(CH // 16):
        ones_v[pl.ds(g * 16, 16)] = ones

    def fcompute(j, carry):
        for g in range(CH // 16):
            sl = pl.ds(g * 16, 16)
            fidx_v[j, sl] = dst_v[j, sl] * 16 + k_v[j, sl]
        return carry

    lax.fori_loop(0, CPW, fcompute, 0)
    plsc.subcore_barrier()

    def fire(j, carry):
        pltpu.async_copy(ones_v, cnt_sh.at[fidx_v.at[j]], sem, add=True)
        return carry

    def drain(j, carry):
        pltpu.make_async_copy(ones_v, cnt_sh.at[fidx_v.at[0]], sem).wait()
        return carry

    lax.fori_loop(0, _CNT_AHEAD, fire, 0)

    def main(j, carry):
        pltpu.make_async_copy(ones_v, cnt_sh.at[fidx_v.at[0]], sem).wait()
        pltpu.async_copy(ones_v, cnt_sh.at[fidx_v.at[j + _CNT_AHEAD]], sem,
                         add=True)
        return carry

    lax.fori_loop(0, CPW - _CNT_AHEAD, main, 0)
    lax.fori_loop(0, _CNT_AHEAD, drain, 0)
    plsc.subcore_barrier()
    pltpu.sync_copy(cnt_sh.at[pl.ds(el0, seg)],
                    out_hbm.at[cid, pl.ds(el0, seg)])


def _make_segsum(with_counts):
    out_type = [jax.ShapeDtypeStruct((NC, N_PAD, D), jnp.float32)]
    scratch = [
        pltpu.VMEM((3, CH), jnp.int32),
        pltpu.VMEM((3, CH), jnp.int32),
        pltpu.VMEM((2, CH, D), jnp.float32),
        pltpu.VMEM_SHARED((N_PAD, D), jnp.float32),
        pltpu.SemaphoreType.DMA,
        pltpu.SemaphoreType.DMA,
        pltpu.SemaphoreType.DMA,
    ]
    if with_counts:
        out_type.append(jax.ShapeDtypeStruct((NC, N_PAD * 16), jnp.float32))
        scratch += [
            pltpu.VMEM((3, CH), jnp.int32),
            pltpu.VMEM((3, CH), jnp.int32),
            pltpu.VMEM((CH,), jnp.float32),
            pltpu.VMEM_SHARED((N_PAD * 16,), jnp.float32),
            pltpu.SemaphoreType.DMA,
        ]

    def body(h_hbm, src_hbm, dst_hbm, z128_hbm, *rest):
        if with_counts:
            (k_hbm, z16_hbm, out_hbm, cnt_hbm, sidx, didx, rows_v, acc_sh,
             gsem, isem, ssem, kidx, fidx, ones_v, cnt_sh, csem) = rest
        else:
            (out_hbm, sidx, didx, rows_v, acc_sh, gsem, isem, ssem) = rest
        cid = lax.axis_index("c")
        sid = lax.axis_index("s")
        wid = sid * NC + cid
        row0 = sid * STRIPE

        def load_idx(j, slot):
            pltpu.async_copy(src_hbm.at[wid, j], sidx.at[slot], isem)
            pltpu.async_copy(dst_hbm.at[wid, j], didx.at[slot], isem)
            if with_counts:
                pltpu.async_copy(k_hbm.at[wid, j], kidx.at[slot], isem)

        def sync_idx(j, slot):
            pltpu.sync_copy(src_hbm.at[wid, j], sidx.at[slot])
            pltpu.sync_copy(dst_hbm.at[wid, j], didx.at[slot])
            if with_counts:
                pltpu.sync_copy(k_hbm.at[wid, j], kidx.at[slot])

        def wait_idx():
            pltpu.make_async_copy(src_hbm.at[wid, 0], sidx.at[0], isem).wait()
            pltpu.make_async_copy(dst_hbm.at[wid, 0], didx.at[0], isem).wait()
            if with_counts:
                pltpu.make_async_copy(k_hbm.at[wid, 0], kidx.at[0], isem).wait()

        sync_idx(0, 0)
        load_idx(1, 1)
        pltpu.async_copy(h_hbm.at[sidx.at[0]], rows_v.at[0], gsem)
        pltpu.sync_copy(z128_hbm.at[pl.ds(row0, STRIPE)],
                        acc_sh.at[pl.ds(row0, STRIPE)])
        if with_counts:
            seg = (N_PAD * 16) // NS
            el0 = sid * seg
            pltpu.sync_copy(z16_hbm.at[pl.ds(el0, seg)],
                            cnt_sh.at[pl.ds(el0, seg)])
            ones = jnp.ones((16,), jnp.float32)
            for g in range(CH // 16):
                ones_v[pl.ds(g * 16, 16)] = ones
        plsc.subcore_barrier()

        def chunk(j, carry):
            p = lax.rem(j, 2)
            s = lax.rem(j, 3)
            pltpu.make_async_copy(h_hbm.at[sidx.at[s]], rows_v.at[p], gsem).wait()

            @pl.when(j >= 1)
            def _():
                pltpu.make_async_copy(rows_v.at[1 - p], acc_sh.at[didx.at[s]],
                                      ssem).wait()

            @pl.when(j < CPW - 1)
            def _():
                wait_idx()
                pltpu.async_copy(h_hbm.at[sidx.at[lax.rem(j + 1, 3)]],
                                 rows_v.at[1 - p], gsem)

                @pl.when(j < CPW - 2)
                def _():
                    load_idx(j + 2, lax.rem(j + 2, 3))

            pltpu.async_copy(rows_v.at[p], acc_sh.at[didx.at[s]], ssem, add=True)
            if with_counts:
                for g in range(CH // 16):
                    sl = pl.ds(g * 16, 16)
                    fidx[s, sl] = didx[s, sl] * 16 + kidx[s, sl]
                pltpu.async_copy(ones_v, cnt_sh.at[fidx.at[s]], csem, add=True)

                @pl.when(j >= 2)
                def _():
                    pltpu.make_async_copy(ones_v, cnt_sh.at[fidx.at[0]],
                                          csem).wait()
            return carry

        lax.fori_loop(0, CPW, chunk, 0)
        pltpu.make_async_copy(rows_v.at[0], acc_sh.at[didx.at[0]], ssem).wait()
        if with_counts:
            pltpu.make_async_copy(ones_v, cnt_sh.at[fidx.at[0]], csem).wait()
            pltpu.make_async_copy(ones_v, cnt_sh.at[fidx.at[0]], csem).wait()
        plsc.subcore_barrier()
        pltpu.sync_copy(acc_sh.at[pl.ds(row0, STRIPE)],
                        out_hbm.at[cid, pl.ds(row0, STRIPE)])
        if with_counts:
            pltpu.sync_copy(cnt_sh.at[pl.ds(el0, seg)],
                            cnt_hbm.at[cid, pl.ds(el0, seg)])

    return pl.kernel(body, out_type=tuple(out_type), mesh=_mesh,
                     scratch_types=scratch)


_sc_segsum_cnt = _make_segsum(True)
_sc_segsum = _make_segsum(False)


def _tc_h0_body(x0_ref, x1_ref, w_ref, o_ref):
    j = lax.broadcasted_iota(jnp.int32, (N, 16), 1)
    oh = (jnp.equal(j, x0_ref[...]) | jnp.equal(j, x1_ref[...] + 8))
    o_ref[...] = jnp.dot(oh.astype(jnp.float32), w_ref[...],
                         preferred_element_type=jnp.float32, precision=lax.Precision.HIGHEST)


_tc_h0 = pl.pallas_call(
    _tc_h0_body,
    out_shape=jax.ShapeDtypeStruct((N, D), jnp.float32),
)


def _tc_layer_body(s_ref, cnt_ref, h_ref, m_ref, w1_ref, b1_ref, w2_ref,
                   b2_ref, g_ref, be_ref, br_ref, o_ref, *, relu_out):
    s = s_ref[0, :N, :] + s_ref[1, :N, :]
    cnt = cnt_ref[0, :N, :] + cnt_ref[1, :N, :]
    aggr = s + h_ref[...] + br_ref[...]
    aggr = aggr + jnp.dot(cnt, m_ref[...], preferred_element_type=jnp.float32, precision=lax.Precision.HIGHEST)
    hid = jnp.maximum(
        jnp.dot(aggr, w1_ref[...], preferred_element_type=jnp.float32)
        + b1_ref[...], 0.0)
    out = jnp.dot(hid, w2_ref[...], preferred_element_type=jnp.float32) + b2_ref[...]
    mean = jnp.mean(out, axis=0, keepdims=True)
    var = jnp.mean((out - mean) ** 2, axis=0, keepdims=True)
    y = g_ref[...] * (out - mean) * lax.rsqrt(var + 1e-5) + be_ref[...]
    if relu_out:
        y = jnp.maximum(y, 0.0)
    o_ref[...] = y


def _tc_layer(relu_out):
    return pl.pallas_call(
        functools.partial(_tc_layer_body, relu_out=relu_out),
        out_shape=jax.ShapeDtypeStruct((N, D), jnp.float32),
    )


_K9_A0 = np.array([0, 0, 0, 1, 1, 1, 2, 2, 2])
_K9_A1 = np.array([0, 1, 2, 0, 1, 2, 0, 1, 2])


def kernel(x, edge_index, edge_attr, x_emb1, x_emb2,
           w1_l0, b1_l0, w2_l0, b2_l0, ee1_l0, ee2_l0, gamma_l0, beta_l0,
           w1_l1, b1_l1, w2_l1, b2_l1, ee1_l1, ee2_l1, gamma_l1, beta_l1):
    src = edge_index[0]
    dst = edge_index[1]
    kcomb = edge_attr[:, 0] * 3 + edge_attr[:, 1]
    pad = E_PAD - E
    src3 = jnp.pad(src.astype(jnp.int32), (0, pad)).reshape(NW, CPW, CH)
    dst3 = jnp.pad(dst.astype(jnp.int32), (0, pad),
                   constant_values=N).reshape(NW, CPW, CH)
    k3 = jnp.pad(kcomb.astype(jnp.int32), (0, pad)).reshape(NW, CPW, CH)

    x0 = x[:, 0:1].astype(jnp.int32)
    x1 = x[:, 1:2].astype(jnp.int32)
    w0 = jnp.zeros((16, D), jnp.float32)
    w0 = w0.at[0:3].set(x_emb1[:3]).at[8:11].set(x_emb2[:3])

    z16 = jnp.zeros((N_PAD * 16,), jnp.float32)
    z128 = jnp.zeros((N_PAD, D), jnp.float32)

    params = [
        (w1_l0, b1_l0, w2_l0, b2_l0, ee1_l0, ee2_l0, gamma_l0, beta_l0),
        (w1_l1, b1_l1, w2_l1, b2_l1, ee1_l1, ee2_l1, gamma_l1, beta_l1),
    ]

    cntp = _sc_counts(dst3, k3, z16).reshape(NC, N_PAD, 16)
    cntp, src3, dst3 = lax.optimization_barrier((cntp, src3, dst3))
    h = _tc_h0(x0, x1, w0)

    for l, (w1, b1, w2, b2, ee1, ee2, g, be) in enumerate(params):
        m16 = jnp.zeros((16, D), jnp.float32).at[:9].set(ee1[_K9_A0] + ee2[_K9_A1])
        br = (ee1[4] + ee2[0]).reshape(1, D)
        (sp,) = _sc_segsum(h, src3, dst3, z128)
        h = _tc_layer(l == 0)(
            sp, cntp, h, m16, w1, b1.reshape(1, 2 * D), w2,
            b2.reshape(1, D), g.reshape(1, D), be.reshape(1, D), br)
    return h


# speedup vs baseline: 1.1198x; 1.1198x over previous
import functools

import jax
import jax.numpy as jnp
import numpy as np
from jax import lax
from jax.experimental import pallas as pl
from jax.experimental.pallas import tpu as pltpu
from jax.experimental.pallas import tpu_sc as plsc

N = 10000
E = 320000
D = 128

NC = 2
NS = 16
NW = NC * NS

CH = 128
CPW = 79
E_PAD = NW * CH * CPW
N_PAD = 10112
STRIPE = N_PAD // NS

_mesh = plsc.VectorSubcoreMesh(core_axis_name="c", subcore_axis_name="s")


_CNT_AHEAD = 8

@functools.partial(
    pl.kernel,
    out_type=jax.ShapeDtypeStruct((NC, N_PAD * 16), jnp.float32),
    mesh=_mesh,
    scratch_types=[
        pltpu.VMEM((CPW, CH), jnp.int32),
        pltpu.VMEM((CPW, CH), jnp.int32),
        pltpu.VMEM((CPW, CH), jnp.int32),
        pltpu.VMEM((CH,), jnp.float32),
        pltpu.VMEM_SHARED((N_PAD * 16,), jnp.float32),
        pltpu.SemaphoreType.DMA,
    ],
)
def _sc_counts(dst_hbm, k_hbm, z16_hbm, out_hbm, dst_v, k_v, fidx_v, ones_v,
               cnt_sh, sem):
    cid = lax.axis_index("c")
    sid = lax.axis_index("s")
    wid = sid * NC + cid
    seg = (N_PAD * 16) // NS
    el0 = sid * seg
    pltpu.sync_copy(dst_hbm.at[wid], dst_v)
    pltpu.sync_copy(k_hbm.at[wid], k_v)
    pltpu.sync_copy(z16_hbm.at[pl.ds(el0, seg)], cnt_sh.at[pl.ds(el0, seg)])
    ones = jnp.ones((16,), jnp.float32)
    for g in range(CH // 16):
        ones_v[pl.ds(g * 16, 16)] = ones

    def fcompute(j, carry):
        for g in range(CH // 16):
            sl = pl.ds(g * 16, 16)
            fidx_v[j, sl] = dst_v[j, sl] * 16 + k_v[j, sl]
        return carry

    lax.fori_loop(0, CPW, fcompute, 0)
    plsc.subcore_barrier()

    def fire(j, carry):
        pltpu.async_copy(ones_v, cnt_sh.at[fidx_v.at[j]], sem, add=True)
        return carry

    def drain(j, carry):
        pltpu.make_async_copy(ones_v, cnt_sh.at[fidx_v.at[0]], sem).wait()
        return carry

    lax.fori_loop(0, _CNT_AHEAD, fire, 0)

    def main(j, carry):
        pltpu.make_async_copy(ones_v, cnt_sh.at[fidx_v.at[0]], sem).wait()
        pltpu.async_copy(ones_v, cnt_sh.at[fidx_v.at[j + _CNT_AHEAD]], sem,
                         add=True)
        return carry

    lax.fori_loop(0, CPW - _CNT_AHEAD, main, 0)
    lax.fori_loop(0, _CNT_AHEAD, drain, 0)
    plsc.subcore_barrier()
    pltpu.sync_copy(cnt_sh.at[pl.ds(el0, seg)],
                    out_hbm.at[cid, pl.ds(el0, seg)])


def _make_segsum(with_counts):
    out_type = [jax.ShapeDtypeStruct((NC, N_PAD, D), jnp.float32)]
    scratch = [
        pltpu.VMEM((3, CH), jnp.int32),
        pltpu.VMEM((3, CH), jnp.int32),
        pltpu.VMEM((2, CH, D), jnp.float32),
        pltpu.VMEM_SHARED((N_PAD, D), jnp.float32),
        pltpu.SemaphoreType.DMA,
        pltpu.SemaphoreType.DMA,
        pltpu.SemaphoreType.DMA,
    ]
    if with_counts:
        out_type.append(jax.ShapeDtypeStruct((NC, N_PAD * 16), jnp.float32))
        scratch += [
            pltpu.VMEM((3, CH), jnp.int32),
            pltpu.VMEM((3, CH), jnp.int32),
            pltpu.VMEM((CH,), jnp.float32),
            pltpu.VMEM_SHARED((N_PAD * 16,), jnp.float32),
            pltpu.SemaphoreType.DMA,
        ]

    def body(h_hbm, src_hbm, dst_hbm, z128_hbm, *rest):
        if with_counts:
            (k_hbm, z16_hbm, out_hbm, cnt_hbm, sidx, didx, rows_v, acc_sh,
             gsem, isem, ssem, kidx, fidx, ones_v, cnt_sh, csem) = rest
        else:
            (out_hbm, sidx, didx, rows_v, acc_sh, gsem, isem, ssem) = rest
        cid = lax.axis_index("c")
        sid = lax.axis_index("s")
        wid = sid * NC + cid
        row0 = sid * STRIPE

        def load_idx(j, slot):
            pltpu.async_copy(src_hbm.at[wid, j], sidx.at[slot], isem)
            pltpu.async_copy(dst_hbm.at[wid, j], didx.at[slot], isem)
            if with_counts:
                pltpu.async_copy(k_hbm.at[wid, j], kidx.at[slot], isem)

        def sync_idx(j, slot):
            pltpu.sync_copy(src_hbm.at[wid, j], sidx.at[slot])
            pltpu.sync_copy(dst_hbm.at[wid, j], didx.at[slot])
            if with_counts:
                pltpu.sync_copy(k_hbm.at[wid, j], kidx.at[slot])

        def wait_idx():
            pltpu.make_async_copy(src_hbm.at[wid, 0], sidx.at[0], isem).wait()
            pltpu.make_async_copy(dst_hbm.at[wid, 0], didx.at[0], isem).wait()
            if with_counts:
                pltpu.make_async_copy(k_hbm.at[wid, 0], kidx.at[0], isem).wait()

        sync_idx(0, 0)
        load_idx(1, 1)
        pltpu.async_copy(h_hbm.at[sidx.at[0]], rows_v.at[0], gsem)
        pltpu.sync_copy(z128_hbm.at[pl.ds(row0, STRIPE)],
                        acc_sh.at[pl.ds(row0, STRIPE)])
        if with_counts:
            seg = (N_PAD * 16) // NS
            el0 = sid * seg
            pltpu.sync_copy(z16_hbm.at[pl.ds(el0, seg)],
                            cnt_sh.at[pl.ds(el0, seg)])
            ones = jnp.ones((16,), jnp.float32)
            for g in range(CH // 16):
                ones_v[pl.ds(g * 16, 16)] = ones
        plsc.subcore_barrier()

        def chunk(j, carry):
            p = lax.rem(j, 2)
            s = lax.rem(j, 3)
            pltpu.make_async_copy(h_hbm.at[sidx.at[s]], rows_v.at[p], gsem).wait()

            @pl.when(j >= 1)
            def _():
                pltpu.make_async_copy(rows_v.at[1 - p], acc_sh.at[didx.at[s]],
                                      ssem).wait()

            @pl.when(j < CPW - 1)
            def _():
                wait_idx()
                pltpu.async_copy(h_hbm.at[sidx.at[lax.rem(j + 1, 3)]],
                                 rows_v.at[1 - p], gsem)

                @pl.when(j < CPW - 2)
                def _():
                    load_idx(j + 2, lax.rem(j + 2, 3))

            pltpu.async_copy(rows_v.at[p], acc_sh.at[didx.at[s]], ssem, add=True)
            if with_counts:
                for g in range(CH // 16):
                    sl = pl.ds(g * 16, 16)
                    fidx[s, sl] = didx[s, sl] * 16 + kidx[s, sl]
                pltpu.async_copy(ones_v, cnt_sh.at[fidx.at[s]], csem, add=True)

                @pl.when(j >= 2)
                def _():
                    pltpu.make_async_copy(ones_v, cnt_sh.at[fidx.at[0]],
                                          csem).wait()
            return carry

        lax.fori_loop(0, CPW, chunk, 0)
        pltpu.make_async_copy(rows_v.at[0], acc_sh.at[didx.at[0]], ssem).wait()
        if with_counts:
            pltpu.make_async_copy(ones_v, cnt_sh.at[fidx.at[0]], csem).wait()
            pltpu.make_async_copy(ones_v, cnt_sh.at[fidx.at[0]], csem).wait()
        plsc.subcore_barrier()
        pltpu.sync_copy(acc_sh.at[pl.ds(row0, STRIPE)],
                        out_hbm.at[cid, pl.ds(row0, STRIPE)])
        if with_counts:
            pltpu.sync_copy(cnt_sh.at[pl.ds(el0, seg)],
                            cnt_hbm.at[cid, pl.ds(el0, seg)])

    return pl.kernel(body, out_type=tuple(out_type), mesh=_mesh,
                     scratch_types=scratch)


_sc_segsum_cnt = _make_segsum(True)
_sc_segsum = _make_segsum(False)


def _tc_h0_body(x0_ref, x1_ref, w_ref, o_ref):
    j = lax.broadcasted_iota(jnp.int32, (N, 16), 1)
    oh = (jnp.equal(j, x0_ref[...]) | jnp.equal(j, x1_ref[...] + 8))
    o_ref[...] = jnp.dot(oh.astype(jnp.float32), w_ref[...],
                         preferred_element_type=jnp.float32, precision=lax.Precision.HIGHEST)


_tc_h0 = pl.pallas_call(
    _tc_h0_body,
    out_shape=jax.ShapeDtypeStruct((N, D), jnp.float32),
)


def _tc_layer_body(s_ref, cnt_ref, h_ref, m_ref, w1_ref, b1_ref, w2_ref,
                   b2_ref, g_ref, be_ref, br_ref, o_ref, *, relu_out):
    s = s_ref[0, :N, :] + s_ref[1, :N, :]
    cnt = cnt_ref[0, :N, :] + cnt_ref[1, :N, :]
    aggr = s + h_ref[...] + br_ref[...]
    aggr = aggr + jnp.dot(cnt, m_ref[...], preferred_element_type=jnp.float32, precision=lax.Precision.HIGHEST)
    hid = jnp.maximum(
        jnp.dot(aggr, w1_ref[...], preferred_element_type=jnp.float32)
        + b1_ref[...], 0.0)
    out = jnp.dot(hid, w2_ref[...], preferred_element_type=jnp.float32) + b2_ref[...]
    mean = jnp.mean(out, axis=0, keepdims=True)
    var = jnp.mean((out - mean) ** 2, axis=0, keepdims=True)
    y = g_ref[...] * (out - mean) * lax.rsqrt(var + 1e-5) + be_ref[...]
    if relu_out:
        y = jnp.maximum(y, 0.0)
    o_ref[...] = y


def _tc_layer(relu_out):
    return pl.pallas_call(
        functools.partial(_tc_layer_body, relu_out=relu_out),
        out_shape=jax.ShapeDtypeStruct((N, D), jnp.float32),
    )


_K9_A0 = np.array([0, 0, 0, 1, 1, 1, 2, 2, 2])
_K9_A1 = np.array([0, 1, 2, 0, 1, 2, 0, 1, 2])


def kernel(x, edge_index, edge_attr, x_emb1, x_emb2,
           w1_l0, b1_l0, w2_l0, b2_l0, ee1_l0, ee2_l0, gamma_l0, beta_l0,
           w1_l1, b1_l1, w2_l1, b2_l1, ee1_l1, ee2_l1, gamma_l1, beta_l1):
    src = edge_index[0]
    dst = edge_index[1]
    kcomb = edge_attr[:, 0] * 3 + edge_attr[:, 1]
    pad = E_PAD - E
    pi = jnp.arange(pad, dtype=jnp.int32)
    src3 = jnp.concatenate([src.astype(jnp.int32), pi]).reshape(NW, CPW, CH)
    dst3 = jnp.concatenate([dst.astype(jnp.int32),
                            N + pi % (N_PAD - N)]).reshape(NW, CPW, CH)
    k3 = jnp.pad(kcomb.astype(jnp.int32), (0, pad)).reshape(NW, CPW, CH)

    x0 = x[:, 0:1].astype(jnp.int32)
    x1 = x[:, 1:2].astype(jnp.int32)
    w0 = jnp.zeros((16, D), jnp.float32)
    w0 = w0.at[0:3].set(x_emb1[:3]).at[8:11].set(x_emb2[:3])

    z16 = jnp.zeros((N_PAD * 16,), jnp.float32)
    z128 = jnp.zeros((N_PAD, D), jnp.float32)

    params = [
        (w1_l0, b1_l0, w2_l0, b2_l0, ee1_l0, ee2_l0, gamma_l0, beta_l0),
        (w1_l1, b1_l1, w2_l1, b2_l1, ee1_l1, ee2_l1, gamma_l1, beta_l1),
    ]

    cntp = _sc_counts(dst3, k3, z16).reshape(NC, N_PAD, 16)
    cntp, src3, dst3 = lax.optimization_barrier((cntp, src3, dst3))
    h = _tc_h0(x0, x1, w0)

    for l, (w1, b1, w2, b2, ee1, ee2, g, be) in enumerate(params):
        m16 = jnp.zeros((16, D), jnp.float32).at[:9].set(ee1[_K9_A0] + ee2[_K9_A1])
        br = (ee1[4] + ee2[0]).reshape(1, D)
        (sp,) = _sc_segsum(h, src3, dst3, z128)
        h = _tc_layer(l == 0)(
            sp, cntp, h, m16, w1, b1.reshape(1, 2 * D), w2,
            b2.reshape(1, D), g.reshape(1, D), be.reshape(1, D), br)
    return h
